# baseline (speedup 1.0000x reference)
import jax
import jax.numpy as jnp
from jax import lax
from jax.experimental import pallas as pl
from jax.experimental.pallas import tpu as pltpu
from jax.experimental.pallas import tpu_sc as plsc

SEQ = 200
DIM = 128
NCORES = 2
NSUB = 16
NW = NCORES * NSUB
ROWS = 1024 * SEQ
RPW = ROWS // NW
CHUNK = 40
NCH = RPW // CHUNK
NPART = SEQ // CHUNK
QSEQ = 4
NQ = NCH // QSEQ
NSLOT = 3
SCALE = 11.313708498984761


def _body(idx_hbm, table_hbm, pos_hbm, out_hbm, idx_v, pos_v, *rest):
    bufs = tuple(tuple(rest[s * QSEQ + j] for j in range(QSEQ))
                 for s in range(NSLOT))
    nb = NSLOT * QSEQ
    gsems = tuple(tuple(rest[nb + s * QSEQ + j] for j in range(QSEQ))
                  for s in range(NSLOT))
    osems = tuple(tuple(rest[2 * nb + s * QSEQ + j] for j in range(QSEQ))
                  for s in range(NSLOT))

    wid = lax.axis_index("s") * NCORES + lax.axis_index("c")
    row0 = wid * RPW

    pltpu.sync_copy(idx_hbm.at[pl.ds(wid * NCH, NCH)], idx_v)

    def quad_gp(q):
        return lax.div(q, NPART), lax.rem(q, NPART)

    def gather_start(q, s):
        g, p = quad_gp(q)
        for j in range(QSEQ):
            cidx = (QSEQ * g + j) * NPART + p
            pltpu.async_copy(table_hbm.at[idx_v.at[cidx]], bufs[s][j],
                             gsems[s][j])

    def gather_wait(q, s):
        g, p = quad_gp(q)
        for j in range(QSEQ):
            cidx = (QSEQ * g + j) * NPART + p
            pltpu.make_async_copy(table_hbm.at[idx_v.at[cidx]], bufs[s][j],
                                  gsems[s][j]).wait()

    def out_start(q, s):
        g, p = quad_gp(q)
        for j in range(QSEQ):
            base = row0 + (QSEQ * g + j) * SEQ + p * CHUNK
            pltpu.async_copy(bufs[s][j], out_hbm.at[pl.ds(base, CHUNK)],
                             osems[s][j])

    def out_wait(q, s):
        g, p = quad_gp(q)
        for j in range(QSEQ):
            base = row0 + (QSEQ * g + j) * SEQ + p * CHUNK
            pltpu.make_async_copy(bufs[s][j], out_hbm.at[pl.ds(base, CHUNK)],
                                  osems[s][j]).wait()

    def compute(q, s):
        _, p = quad_gp(q)
        off = p * CHUNK

        @plsc.parallel_loop(0, CHUNK, unroll=2)
        def row_fma(r):
            pr = off + r
            for k in range(DIM // 16):
                sl = pl.ds(k * 16, 16)
                pv = pos_v[pr, sl]
                for j in range(QSEQ):
                    bufs[s][j][r, sl] = bufs[s][j][r, sl] * SCALE + pv

    gather_start(0, 0)
    gather_start(1, 1)
    pltpu.sync_copy(pos_hbm, pos_v)

    gather_wait(0, 0)
    compute(0, 0)
    out_start(0, 0)
    gather_start(2, 2)

    def cycle(t, _):
        for u in range(NSLOT):
            q = 1 + t * NSLOT + u
            s = (1 + u) % NSLOT
            gather_wait(q, s)
            compute(q, s)
            out_start(q, s)
            out_wait(q - 1, (u % NSLOT))
            gather_start(q + 2, (u % NSLOT))
        return _

    lax.fori_loop(0, (NQ - 4) // NSLOT, cycle, 0)

    q = NQ - 3
    gather_wait(q, q % NSLOT)
    compute(q, q % NSLOT)
    out_start(q, q % NSLOT)
    out_wait(q - 1, (q - 1) % NSLOT)
    gather_start(q + 2, (q + 2) % NSLOT)

    for q in (NQ - 2, NQ - 1):
        gather_wait(q, q % NSLOT)
        compute(q, q % NSLOT)
        out_start(q, q % NSLOT)
        out_wait(q - 1, (q - 1) % NSLOT)
    out_wait(NQ - 1, (NQ - 1) % NSLOT)


def kernel(inputs, table, pos_enc):
    flat_idx = inputs.reshape(ROWS // CHUNK, CHUNK).astype(jnp.int32)

    mesh = plsc.VectorSubcoreMesh(core_axis_name="c", subcore_axis_name="s")
    run = pl.kernel(
        _body,
        mesh=mesh,
        out_type=jax.ShapeDtypeStruct((ROWS, DIM), jnp.float32),
        scratch_types=[
            pltpu.VMEM((NCH, CHUNK), jnp.int32),
            pltpu.VMEM((SEQ, DIM), jnp.float32),
        ] + [pltpu.VMEM((CHUNK, DIM), jnp.float32)] * (NSLOT * QSEQ)
          + [pltpu.SemaphoreType.DMA] * (2 * NSLOT * QSEQ),
    )
    out = run(flat_idx, table, pos_enc)
    return out.reshape(1024, SEQ, DIM)


# speedup vs baseline: 1.1708x; 1.1708x over previous
import jax
import jax.numpy as jnp
from jax import lax
from jax.experimental import pallas as pl
from jax.experimental.pallas import tpu as pltpu
from jax.experimental.pallas import tpu_sc as plsc

SEQ = 200
DIM = 128
NCORES = 2
NSUB = 16
NW = NCORES * NSUB
ROWS = 1024 * SEQ
RPW = ROWS // NW
CHUNK = 40
NCH = RPW // CHUNK
NPART = SEQ // CHUNK
QSEQ = 4
NQ = NCH // QSEQ
NSLOT = 2
SCALE = 11.313708498984761


def _body(idx_hbm, table_hbm, pos_hbm, out_hbm, idx_v, pos_v, *rest):
    nb = NSLOT * QSEQ
    ins = tuple(tuple(rest[s * QSEQ + j] for j in range(QSEQ))
                for s in range(NSLOT))
    outs = tuple(tuple(rest[nb + s * QSEQ + j] for j in range(QSEQ))
                 for s in range(NSLOT))
    gsems = tuple(tuple(rest[2 * nb + s * QSEQ + j] for j in range(QSEQ))
                  for s in range(NSLOT))
    osems = tuple(tuple(rest[3 * nb + s * QSEQ + j] for j in range(QSEQ))
                  for s in range(NSLOT))

    wid = lax.axis_index("s") * NCORES + lax.axis_index("c")
    row0 = wid * RPW

    pltpu.sync_copy(idx_hbm.at[pl.ds(wid * NCH, NCH)], idx_v)

    def quad_gp(q):
        return lax.div(q, NPART), lax.rem(q, NPART)

    def gather_start(q, s):
        g, p = quad_gp(q)
        for j in range(QSEQ):
            cidx = (QSEQ * g + j) * NPART + p
            pltpu.async_copy(table_hbm.at[idx_v.at[cidx]], ins[s][j],
                             gsems[s][j])

    def gather_wait(q, s):
        g, p = quad_gp(q)
        for j in range(QSEQ):
            cidx = (QSEQ * g + j) * NPART + p
            pltpu.make_async_copy(table_hbm.at[idx_v.at[cidx]], ins[s][j],
                                  gsems[s][j]).wait()

    def out_start(q, s):
        g, p = quad_gp(q)
        for j in range(QSEQ):
            base = row0 + (QSEQ * g + j) * SEQ + p * CHUNK
            pltpu.async_copy(outs[s][j], out_hbm.at[pl.ds(base, CHUNK)],
                             osems[s][j])

    def out_wait(q, s):
        g, p = quad_gp(q)
        for j in range(QSEQ):
            base = row0 + (QSEQ * g + j) * SEQ + p * CHUNK
            pltpu.make_async_copy(outs[s][j], out_hbm.at[pl.ds(base, CHUNK)],
                                  osems[s][j]).wait()

    def compute(q, s):
        _, p = quad_gp(q)
        off = p * CHUNK

        @plsc.parallel_loop(0, CHUNK, unroll=2)
        def row_fma(r):
            pr = off + r
            for k in range(DIM // 16):
                sl = pl.ds(k * 16, 16)
                pv = pos_v[pr, sl]
                for j in range(QSEQ):
                    outs[s][j][r, sl] = ins[s][j][r, sl] * SCALE + pv

    gather_start(0, 0)
    gather_start(1, 1)
    pltpu.sync_copy(pos_hbm, pos_v)

    for q in range(NSLOT):
        gather_wait(q, q)
        compute(q, q)
        gather_start(q + NSLOT, q)
        out_start(q, q)

    def cycle(t, _):
        for u in range(NSLOT):
            q = NSLOT + t * NSLOT + u
            gather_wait(q, u)
            out_wait(q - NSLOT, u)
            compute(q, u)
            gather_start(q + NSLOT, u)
            out_start(q, u)
        return _

    lax.fori_loop(0, (NQ - 2 * NSLOT) // NSLOT, cycle, 0)

    for u in range(NSLOT):
        q = NQ - NSLOT + u
        gather_wait(q, u)
        out_wait(q - NSLOT, u)
        compute(q, u)
        out_start(q, u)
    for u in range(NSLOT):
        out_wait(NQ - NSLOT + u, u)


def kernel(inputs, table, pos_enc):
    flat_idx = inputs.reshape(ROWS // CHUNK, CHUNK).astype(jnp.int32)

    mesh = plsc.VectorSubcoreMesh(core_axis_name="c", subcore_axis_name="s")
    run = pl.kernel(
        _body,
        mesh=mesh,
        out_type=jax.ShapeDtypeStruct((ROWS, DIM), jnp.float32),
        scratch_types=[
            pltpu.VMEM((NCH, CHUNK), jnp.int32),
            pltpu.VMEM((SEQ, DIM), jnp.float32),
        ] + [pltpu.VMEM((CHUNK, DIM), jnp.float32)] * (2 * NSLOT * QSEQ)
          + [pltpu.SemaphoreType.DMA] * (2 * NSLOT * QSEQ),
    )
    out = run(flat_idx, table, pos_enc)
    return out.reshape(1024, SEQ, DIM)


# speedup vs baseline: 1.2166x; 1.0391x over previous
import jax
import jax.numpy as jnp
from jax import lax
from jax.experimental import pallas as pl
from jax.experimental.pallas import tpu as pltpu
from jax.experimental.pallas import tpu_sc as plsc

SEQ = 200
DIM = 128
NCORES = 2
NSUB = 16
NW = NCORES * NSUB
ROWS = 1024 * SEQ
RPW = ROWS // NW
CHUNK = 40
NCH = RPW // CHUNK
NPART = SEQ // CHUNK
QSEQ = 4
NQ = NCH // QSEQ
NSLOT = 2
SCALE = 11.313708498984761


def _body(idx_hbm, table_hbm, pos_hbm, out_hbm, idx_v, pos_v, *rest):
    nb = NSLOT * QSEQ
    ins = tuple(tuple(rest[s * QSEQ + j] for j in range(QSEQ))
                for s in range(NSLOT))
    outs = tuple(tuple(rest[nb + s * QSEQ + j] for j in range(QSEQ))
                 for s in range(NSLOT))
    gsems = tuple(tuple(rest[2 * nb + s * QSEQ + j] for j in range(QSEQ))
                  for s in range(NSLOT))
    osems = tuple(tuple(rest[3 * nb + s * QSEQ + j] for j in range(QSEQ))
                  for s in range(NSLOT))

    wid = lax.axis_index("s") * NCORES + lax.axis_index("c")
    row0 = wid * RPW

    pltpu.sync_copy(idx_hbm.at[pl.ds(wid * NCH, NCH)], idx_v)

    def quad_gp(q):
        return lax.div(q, NPART), lax.rem(q, NPART)

    def gather_start(q, s):
        g, p = quad_gp(q)
        for j in range(QSEQ):
            cidx = (QSEQ * g + j) * NPART + p
            pltpu.async_copy(table_hbm.at[idx_v.at[cidx]], ins[s][j],
                             gsems[s][j])

    def gather_wait(q, s):
        g, p = quad_gp(q)
        for j in range(QSEQ):
            cidx = (QSEQ * g + j) * NPART + p
            pltpu.make_async_copy(table_hbm.at[idx_v.at[cidx]], ins[s][j],
                                  gsems[s][j]).wait()

    def out_start(q, s):
        g, p = quad_gp(q)
        for j in range(QSEQ):
            base = row0 + (QSEQ * g + j) * SEQ + p * CHUNK
            pltpu.async_copy(outs[s][j], out_hbm.at[pl.ds(base, CHUNK)],
                             osems[s][j])

    def out_wait(q, s):
        g, p = quad_gp(q)
        for j in range(QSEQ):
            base = row0 + (QSEQ * g + j) * SEQ + p * CHUNK
            pltpu.make_async_copy(outs[s][j], out_hbm.at[pl.ds(base, CHUNK)],
                                  osems[s][j]).wait()

    def compute(q, s):
        _, p = quad_gp(q)
        off = p * CHUNK

        @plsc.parallel_loop(0, CHUNK, unroll=2)
        def row_fma(r):
            pr = off + r
            for k in range(DIM // 16):
                sl = pl.ds(k * 16, 16)
                pv = pos_v[pr, sl]
                for j in range(QSEQ):
                    outs[s][j][r, sl] = ins[s][j][r, sl] * SCALE + pv

    gather_start(0, 0)
    gather_start(1, 1)
    pltpu.sync_copy(pos_hbm, pos_v)

    def cycle(t, carry):
        for u in range(NSLOT):
            q = t * NSLOT + u
            gather_wait(q, u)

            @pl.when(q >= NSLOT)
            def _ow():
                out_wait(q - NSLOT, u)

            compute(q, u)

            @pl.when(q < NQ - NSLOT)
            def _gs():
                gather_start(q + NSLOT, u)

            out_start(q, u)
        return carry

    lax.fori_loop(0, NQ // NSLOT, cycle, 0)
    for u in range(NSLOT):
        out_wait(NQ - NSLOT + u, u)


def kernel(inputs, table, pos_enc):
    flat_idx = inputs.reshape(ROWS // CHUNK, CHUNK).astype(jnp.int32)

    mesh = plsc.VectorSubcoreMesh(core_axis_name="c", subcore_axis_name="s")
    run = pl.kernel(
        _body,
        mesh=mesh,
        out_type=jax.ShapeDtypeStruct((ROWS, DIM), jnp.float32),
        scratch_types=[
            pltpu.VMEM((NCH, CHUNK), jnp.int32),
            pltpu.VMEM((SEQ, DIM), jnp.float32),
        ] + [pltpu.VMEM((CHUNK, DIM), jnp.float32)] * (2 * NSLOT * QSEQ)
          + [pltpu.SemaphoreType.DMA] * (2 * NSLOT * QSEQ),
    )
    out = run(flat_idx, table, pos_enc)
    return out.reshape(1024, SEQ, DIM)


# speedup vs baseline: 1.2296x; 1.0107x over previous
import jax
import jax.numpy as jnp
from jax import lax
from jax.experimental import pallas as pl
from jax.experimental.pallas import tpu as pltpu
from jax.experimental.pallas import tpu_sc as plsc

SEQ = 200
DIM = 128
NCORES = 2
NSUB = 16
NW = NCORES * NSUB
ROWS = 1024 * SEQ
RPW = ROWS // NW
CHUNK = 40
NCH = RPW // CHUNK
NPART = SEQ // CHUNK
QSEQ = 4
NQ = NCH // QSEQ
NSLOT = 2
SCALE = 11.313708498984761


def _body(idx_hbm, table_hbm, pos_hbm, out_hbm, idx_v, pos_v, *rest):
    nb = NSLOT * QSEQ
    ins = tuple(tuple(rest[s * QSEQ + j] for j in range(QSEQ))
                for s in range(NSLOT))
    outs = tuple(tuple(rest[nb + s * QSEQ + j] for j in range(QSEQ))
                 for s in range(NSLOT))
    gsems = tuple(tuple(rest[2 * nb + s * QSEQ + j] for j in range(QSEQ))
                  for s in range(NSLOT))
    osems = tuple(tuple(rest[3 * nb + s * QSEQ + j] for j in range(QSEQ))
                  for s in range(NSLOT))

    wid = lax.axis_index("s") * NCORES + lax.axis_index("c")
    row0 = wid * RPW

    pltpu.sync_copy(idx_hbm.at[pl.ds(wid * NCH, NCH)], idx_v)

    def quad_gp(q):
        return lax.div(q, NPART), lax.rem(q, NPART)

    def gather_start(q, s):
        g, p = quad_gp(q)
        for j in range(QSEQ):
            cidx = (QSEQ * g + j) * NPART + p
            pltpu.async_copy(table_hbm.at[idx_v.at[cidx]], ins[s][j],
                             gsems[s][j])

    def gather_wait(q, s):
        g, p = quad_gp(q)
        for j in range(QSEQ):
            cidx = (QSEQ * g + j) * NPART + p
            pltpu.make_async_copy(table_hbm.at[idx_v.at[cidx]], ins[s][j],
                                  gsems[s][j]).wait()

    def out_start(q, s):
        g, p = quad_gp(q)
        for j in range(QSEQ):
            base = row0 + (QSEQ * g + j) * SEQ + p * CHUNK
            pltpu.async_copy(outs[s][j], out_hbm.at[pl.ds(base, CHUNK)],
                             osems[s][j])

    def out_wait(q, s):
        g, p = quad_gp(q)
        for j in range(QSEQ):
            base = row0 + (QSEQ * g + j) * SEQ + p * CHUNK
            pltpu.make_async_copy(outs[s][j], out_hbm.at[pl.ds(base, CHUNK)],
                                  osems[s][j]).wait()

    def compute(q, s):
        _, p = quad_gp(q)
        off = p * CHUNK

        @plsc.parallel_loop(0, CHUNK, unroll=3)
        def row_fma(r):
            pr = off + r
            for k in range(DIM // 16):
                sl = pl.ds(k * 16, 16)
                pv = pos_v[pr, sl]
                for j in range(QSEQ):
                    outs[s][j][r, sl] = ins[s][j][r, sl] * SCALE + pv

    gather_start(0, 0)
    gather_start(1, 1)
    pltpu.sync_copy(pos_hbm, pos_v)

    def cycle(t, carry):
        for u in range(NSLOT):
            q = t * NSLOT + u
            gather_wait(q, u)

            @pl.when(q >= NSLOT)
            def _ow():
                out_wait(q - NSLOT, u)

            compute(q, u)

            @pl.when(q < NQ - NSLOT)
            def _gs():
                gather_start(q + NSLOT, u)

            out_start(q, u)
        return carry

    lax.fori_loop(0, NQ // NSLOT, cycle, 0)
    for u in range(NSLOT):
        out_wait(NQ - NSLOT + u, u)


def kernel(inputs, table, pos_enc):
    flat_idx = inputs.reshape(ROWS // CHUNK, CHUNK).astype(jnp.int32)

    mesh = plsc.VectorSubcoreMesh(core_axis_name="c", subcore_axis_name="s")
    run = pl.kernel(
        _body,
        mesh=mesh,
        out_type=jax.ShapeDtypeStruct((ROWS, DIM), jnp.float32),
        scratch_types=[
            pltpu.VMEM((NCH, CHUNK), jnp.int32),
            pltpu.VMEM((SEQ, DIM), jnp.float32),
        ] + [pltpu.VMEM((CHUNK, DIM), jnp.float32)] * (2 * NSLOT * QSEQ)
          + [pltpu.SemaphoreType.DMA] * (2 * NSLOT * QSEQ),
    )
    out = run(flat_idx, table, pos_enc)
    return out.reshape(1024, SEQ, DIM)


# speedup vs baseline: 1.2339x; 1.0035x over previous
import jax
import jax.numpy as jnp
from jax import lax
from jax.experimental import pallas as pl
from jax.experimental.pallas import tpu as pltpu
from jax.experimental.pallas import tpu_sc as plsc

SEQ = 200
DIM = 128
NCORES = 2
NSUB = 16
NW = NCORES * NSUB
ROWS = 1024 * SEQ
RPW = ROWS // NW
CHUNK = 40
NCH = RPW // CHUNK
NPART = SEQ // CHUNK
QSEQ = 4
NQ = NCH // QSEQ
NSLOT = 2
SCALE = 11.313708498984761


def _body(idx_hbm, table_hbm, pos_hbm, out_hbm, idx_v, pos_v, *rest):
    nb = NSLOT * QSEQ
    ins = tuple(tuple(rest[s * QSEQ + j] for j in range(QSEQ))
                for s in range(NSLOT))
    outs = tuple(tuple(rest[nb + s * QSEQ + j] for j in range(QSEQ))
                 for s in range(NSLOT))
    gsems = tuple(tuple(rest[2 * nb + s * QSEQ + j] for j in range(QSEQ))
                  for s in range(NSLOT))
    osems = tuple(tuple(rest[3 * nb + s * QSEQ + j] for j in range(QSEQ))
                  for s in range(NSLOT))

    wid = lax.axis_index("s") * NCORES + lax.axis_index("c")
    row0 = wid * RPW

    pltpu.sync_copy(idx_hbm.at[pl.ds(wid * NCH, NCH)], idx_v)

    def quad_gp(q):
        return lax.div(q, NPART), lax.rem(q, NPART)

    def gather_start(q, s):
        g, p = quad_gp(q)
        for j in range(QSEQ):
            cidx = (QSEQ * g + j) * NPART + p
            pltpu.async_copy(table_hbm.at[idx_v.at[cidx]], ins[s][j],
                             gsems[s][j])

    def gather_wait(q, s):
        g, p = quad_gp(q)
        for j in range(QSEQ):
            cidx = (QSEQ * g + j) * NPART + p
            pltpu.make_async_copy(table_hbm.at[idx_v.at[cidx]], ins[s][j],
                                  gsems[s][j]).wait()

    def out_start(q, s):
        g, p = quad_gp(q)
        for j in range(QSEQ):
            base = row0 + (QSEQ * g + j) * SEQ + p * CHUNK
            pltpu.async_copy(outs[s][j], out_hbm.at[pl.ds(base, CHUNK)],
                             osems[s][j])

    def out_wait(q, s):
        g, p = quad_gp(q)
        for j in range(QSEQ):
            base = row0 + (QSEQ * g + j) * SEQ + p * CHUNK
            pltpu.make_async_copy(outs[s][j], out_hbm.at[pl.ds(base, CHUNK)],
                                  osems[s][j]).wait()

    def compute(q, s):
        _, p = quad_gp(q)
        off = p * CHUNK

        @plsc.parallel_loop(0, CHUNK, unroll=1)
        def row_fma(r):
            pr = off + r
            for k in range(DIM // 16):
                sl = pl.ds(k * 16, 16)
                pv = pos_v[pr, sl]
                for j in range(QSEQ):
                    outs[s][j][r, sl] = ins[s][j][r, sl] * SCALE + pv

    gather_start(0, 0)
    gather_start(1, 1)
    pltpu.sync_copy(pos_hbm, pos_v)

    def cycle(t, carry):
        for u in range(NSLOT):
            q = t * NSLOT + u
            gather_wait(q, u)

            @pl.when(q >= NSLOT)
            def _ow():
                out_wait(q - NSLOT, u)

            compute(q, u)

            @pl.when(q < NQ - NSLOT)
            def _gs():
                gather_start(q + NSLOT, u)

            out_start(q, u)
        return carry

    lax.fori_loop(0, NQ // NSLOT, cycle, 0)
    for u in range(NSLOT):
        out_wait(NQ - NSLOT + u, u)


def kernel(inputs, table, pos_enc):
    flat_idx = inputs.reshape(ROWS // CHUNK, CHUNK).astype(jnp.int32)

    mesh = plsc.VectorSubcoreMesh(core_axis_name="c", subcore_axis_name="s")
    run = pl.kernel(
        _body,
        mesh=mesh,
        out_type=jax.ShapeDtypeStruct((ROWS, DIM), jnp.float32),
        scratch_types=[
            pltpu.VMEM((NCH, CHUNK), jnp.int32),
            pltpu.VMEM((SEQ, DIM), jnp.float32),
        ] + [pltpu.VMEM((CHUNK, DIM), jnp.float32)] * (2 * NSLOT * QSEQ)
          + [pltpu.SemaphoreType.DMA] * (2 * NSLOT * QSEQ),
    )
    out = run(flat_idx, table, pos_enc)
    return out.reshape(1024, SEQ, DIM)
